# Initial kernel scaffold; baseline (speedup 1.0000x reference)
import functools

import jax
import jax.numpy as jnp
from jax import lax
from jax.experimental import pallas as pl
from jax.experimental.pallas import tpu as pltpu
from jax.experimental.pallas import tpu_sc as plsc

N_NODES = 10000
N_EDGES = 320000
D = 128

NC = 2
NS = 16
NW = NC * NS
EPW = N_EDGES // NW
CH = 80
CHUNKS = EPW // CH
RPT = N_NODES // NS
ZR = 125

_mesh = plsc.VectorSubcoreMesh(core_axis_name="c", subcore_axis_name="s")


@functools.partial(
    pl.kernel,
    out_type=jax.ShapeDtypeStruct((NC, N_NODES, D), jnp.float32),
    mesh=_mesh,
    scratch_types=[
        pltpu.VMEM((CH,), jnp.int32),
        pltpu.VMEM((CH,), jnp.int32),
        pltpu.VMEM((CH, D), jnp.float32),
        pltpu.VMEM((ZR, D), jnp.float32),
        pltpu.VMEM_SHARED((N_NODES, D), jnp.float32),
    ],
)
def _segment_sum_sc(x_hbm, src_hbm, dst_hbm, out_hbm, sidx, didx, buf, zbuf, acc):
    cid = lax.axis_index("c")
    sid = lax.axis_index("s")
    wid = sid * NC + cid

    @pl.loop(0, ZR)
    def _(i):
        for j in range(D // 16):
            zbuf.at[pl.ds(i, 1), pl.ds(j * 16, 16)][...] = jnp.zeros(
                (1, 16), jnp.float32)

    for j in range(RPT // ZR):
        pltpu.sync_copy(zbuf, acc.at[pl.ds(sid * RPT + j * ZR, ZR)])
    plsc.subcore_barrier()

    @pl.loop(0, CHUNKS)
    def _(g):
        base = wid * EPW + g * CH
        pltpu.sync_copy(src_hbm.at[pl.ds(base, CH)], sidx)
        pltpu.sync_copy(dst_hbm.at[pl.ds(base, CH)], didx)
        pltpu.sync_copy(x_hbm.at[sidx], buf)
        pltpu.sync_copy(buf, acc.at[didx], add=True)

    plsc.subcore_barrier()

    for j in range(RPT // ZR):
        r = sid * RPT + j * ZR
        pltpu.sync_copy(acc.at[pl.ds(r, ZR)], out_hbm.at[cid, pl.ds(r, ZR)])


def _linear_body(x_ref, s0_ref, s1_ref, w_ref, b_ref, a_ref, bb_ref):
    w1 = w_ref[0:D, :]
    w2 = w_ref[D:2 * D, :]
    a_ref[...] = jnp.dot(x_ref[...], w1,
                         preferred_element_type=jnp.float32) + b_ref[...]
    bb_ref[...] = jnp.dot(s0_ref[...] + s1_ref[...], w2,
                          preferred_element_type=jnp.float32)


_ROWS_BLK = 1000


def _linear_tc(x, s0, s1, W, b):
    grid = (N_NODES // _ROWS_BLK,)
    blk = pl.BlockSpec((_ROWS_BLK, D), lambda i: (i, 0))
    return pl.pallas_call(
        _linear_body,
        grid=grid,
        in_specs=[blk, blk, blk,
                  pl.BlockSpec((2 * D, D), lambda i: (0, 0)),
                  pl.BlockSpec((1, D), lambda i: (0, 0))],
        out_specs=[blk, blk],
        out_shape=[jax.ShapeDtypeStruct((N_NODES, D), jnp.float32)] * 2,
    )(x, s0, s1, W, b.reshape(1, D))


@functools.partial(
    pl.kernel,
    out_type=jax.ShapeDtypeStruct((N_EDGES, D), jnp.float32),
    mesh=_mesh,
    scratch_types=[
        pltpu.VMEM((CH,), jnp.int32),
        pltpu.VMEM((CH,), jnp.int32),
        pltpu.VMEM((CH, D), jnp.float32),
        pltpu.VMEM((CH, D), jnp.float32),
    ],
)
def _edge_combine_sc(a_hbm, b_hbm, src_hbm, dst_hbm, out_hbm,
                     sidx, didx, bufa, bufb):
    cid = lax.axis_index("c")
    sid = lax.axis_index("s")
    wid = sid * NC + cid

    @pl.loop(0, CHUNKS)
    def _(g):
        base = wid * EPW + g * CH
        pltpu.sync_copy(src_hbm.at[pl.ds(base, CH)], sidx)
        pltpu.sync_copy(dst_hbm.at[pl.ds(base, CH)], didx)
        pltpu.sync_copy(a_hbm.at[sidx], bufa)
        pltpu.sync_copy(b_hbm.at[didx], bufb)

        @pl.loop(0, CH)
        def _(r):
            for j in range(D // 16):
                sl = (pl.ds(r, 1), pl.ds(j * 16, 16))
                bufa.at[sl][...] += bufb.at[sl][...]

        pltpu.sync_copy(bufa, out_hbm.at[pl.ds(base, CH)])


def kernel(x, edge_index, W, b):
    src = edge_index[0].astype(jnp.int32)
    dst = edge_index[1].astype(jnp.int32)
    s_part = _segment_sum_sc(x, src, dst)
    a, bb = _linear_tc(x, s_part[0], s_part[1], W, b)
    return _edge_combine_sc(a, bb, src, dst)


# speedup vs baseline: 2.4207x; 2.4207x over previous
import functools

import jax
import jax.numpy as jnp
from jax import lax
from jax.experimental import pallas as pl
from jax.experimental.pallas import tpu as pltpu
from jax.experimental.pallas import tpu_sc as plsc

N_NODES = 10000
N_EDGES = 320000
D = 128

NC = 2
NS = 16
NW = NC * NS
EPW = N_EDGES // NW
CH = 80
CHUNKS = EPW // CH
RPT = 624
RPT_LAST = N_NODES - (NS - 1) * RPT
ZB = 16

_mesh = plsc.VectorSubcoreMesh(core_axis_name="c", subcore_axis_name="s")
_mesh1 = plsc.VectorSubcoreMesh(core_axis_name="c", subcore_axis_name="s",
                                num_cores=1)


EPW1 = N_EDGES // NS
CHUNKS1 = EPW1 // CH


@functools.partial(
    pl.kernel,
    out_type=jax.ShapeDtypeStruct((N_NODES, D), jnp.float32),
    mesh=_mesh1,
    scratch_types=[
        pltpu.VMEM((CH,), jnp.int32),
        pltpu.VMEM((CH,), jnp.int32),
        pltpu.VMEM((CH, D), jnp.float32),
        pltpu.VMEM((ZB, D), jnp.float32),
        pltpu.VMEM_SHARED((N_NODES, D), jnp.float32),
    ],
)
def _segment_sum_sc(x_hbm, src_hbm, dst_hbm, out_hbm, sidx, didx, buf, zbuf, acc):
    sid = lax.axis_index("s")

    @pl.loop(0, ZB)
    def _(i):
        for j in range(D // 16):
            zbuf.at[pl.ds(i, 1), pl.ds(j * 16, 16)][...] = jnp.zeros(
                (1, 16), jnp.float32)

    @pl.when(sid < NS - 1)
    def _():
        @pl.loop(0, RPT // ZB)
        def _(k):
            pltpu.sync_copy(zbuf, acc.at[pl.ds(sid * RPT + k * ZB, ZB)])

    @pl.when(sid == NS - 1)
    def _():
        @pl.loop(0, RPT_LAST // ZB)
        def _(k):
            pltpu.sync_copy(zbuf, acc.at[pl.ds((NS - 1) * RPT + k * ZB, ZB)])

    plsc.subcore_barrier()

    @pl.loop(0, CHUNKS1)
    def _(g):
        base = sid * EPW1 + g * CH
        pltpu.sync_copy(src_hbm.at[pl.ds(base, CH)], sidx)
        pltpu.sync_copy(dst_hbm.at[pl.ds(base, CH)], didx)
        pltpu.sync_copy(x_hbm.at[sidx], buf)
        pltpu.sync_copy(buf, acc.at[didx], add=True)

    plsc.subcore_barrier()

    @pl.when(sid < NS - 1)
    def _():
        r = sid * RPT
        pltpu.sync_copy(acc.at[pl.ds(r, RPT)], out_hbm.at[pl.ds(r, RPT)])

    @pl.when(sid == NS - 1)
    def _():
        r = (NS - 1) * RPT
        pltpu.sync_copy(acc.at[pl.ds(r, RPT_LAST)],
                        out_hbm.at[pl.ds(r, RPT_LAST)])


def _linear_body(x_ref, s_ref, w_ref, b_ref, a_ref, bb_ref):
    w1 = w_ref[0:D, :]
    w2 = w_ref[D:2 * D, :]
    a_ref[...] = jnp.dot(x_ref[...], w1,
                         preferred_element_type=jnp.float32) + b_ref[...]
    bb_ref[...] = jnp.dot(s_ref[...], w2,
                          preferred_element_type=jnp.float32)


_ROWS_BLK = 1000


def _linear_tc(x, s, W, b):
    grid = (N_NODES // _ROWS_BLK,)
    blk = pl.BlockSpec((_ROWS_BLK, D), lambda i: (i, 0))
    return pl.pallas_call(
        _linear_body,
        grid=grid,
        in_specs=[blk, blk,
                  pl.BlockSpec((2 * D, D), lambda i: (0, 0)),
                  pl.BlockSpec((1, D), lambda i: (0, 0))],
        out_specs=[blk, blk],
        out_shape=[jax.ShapeDtypeStruct((N_NODES, D), jnp.float32)] * 2,
    )(x, s, W, b.reshape(1, D))


@functools.partial(
    pl.kernel,
    out_type=jax.ShapeDtypeStruct((N_EDGES, D), jnp.float32),
    mesh=_mesh,
    scratch_types=[
        pltpu.VMEM((CH,), jnp.int32),
        pltpu.VMEM((CH,), jnp.int32),
        pltpu.VMEM((CH, D), jnp.float32),
        pltpu.VMEM((CH, D), jnp.float32),
    ],
)
def _edge_combine_sc(a_hbm, b_hbm, src_hbm, dst_hbm, out_hbm,
                     sidx, didx, bufa, bufb):
    cid = lax.axis_index("c")
    sid = lax.axis_index("s")
    wid = sid * NC + cid

    @pl.loop(0, CHUNKS)
    def _(g):
        base = wid * EPW + g * CH
        pltpu.sync_copy(src_hbm.at[pl.ds(base, CH)], sidx)
        pltpu.sync_copy(dst_hbm.at[pl.ds(base, CH)], didx)
        pltpu.sync_copy(a_hbm.at[sidx], bufa)
        pltpu.sync_copy(b_hbm.at[didx], bufb)

        @pl.loop(0, CH)
        def _(r):
            for j in range(D // 16):
                sl = (pl.ds(r, 1), pl.ds(j * 16, 16))
                bufa.at[sl][...] += bufb.at[sl][...]

        pltpu.sync_copy(bufa, out_hbm.at[pl.ds(base, CH)])


def kernel(x, edge_index, W, b):
    src = edge_index[0].astype(jnp.int32)
    dst = edge_index[1].astype(jnp.int32)
    s = _segment_sum_sc(x, src, dst)
    a, bb = _linear_tc(x, s, W, b)
    return _edge_combine_sc(a, bb, src, dst)


# speedup vs baseline: 6.2854x; 2.5966x over previous
import functools

import jax
import jax.numpy as jnp
from jax import lax
from jax.experimental import pallas as pl
from jax.experimental.pallas import tpu as pltpu
from jax.experimental.pallas import tpu_sc as plsc

N_NODES = 10000
N_EDGES = 320000
D = 128

NC = 2
NS = 16
NW = NC * NS
EPW = N_EDGES // NW
CH = 80
CHUNKS = EPW // CH
RPT = 624
RPT_LAST = N_NODES - (NS - 1) * RPT
ZB = 16

_mesh = plsc.VectorSubcoreMesh(core_axis_name="c", subcore_axis_name="s")
_mesh1 = plsc.VectorSubcoreMesh(core_axis_name="c", subcore_axis_name="s",
                                num_cores=1)

EPW1 = N_EDGES // NS
H1 = 5
HC = EPW1 // (H1 * CH)


@functools.partial(
    pl.kernel,
    out_type=jax.ShapeDtypeStruct((N_NODES, D), jnp.float32),
    mesh=_mesh1,
    scratch_types=[
        pltpu.VMEM((HC, CH), jnp.int32),
        pltpu.VMEM((HC, CH), jnp.int32),
        pltpu.VMEM((2, CH, D), jnp.float32),
        pltpu.VMEM((ZB, D), jnp.float32),
        pltpu.VMEM_SHARED((N_NODES, D), jnp.float32),
        pltpu.SemaphoreType.DMA,
        pltpu.SemaphoreType.DMA,
    ],
)
def _segment_sum_sc(x_hbm, src_hbm, dst_hbm, out_hbm,
                    sidx, didx, bufx, zbuf, acc, sg0, sg1):
    sid = lax.axis_index("s")
    sg = (sg0, sg1)

    @pl.loop(0, ZB)
    def _(i):
        for j in range(D // 16):
            zbuf.at[pl.ds(i, 1), pl.ds(j * 16, 16)][...] = jnp.zeros(
                (1, 16), jnp.float32)

    @pl.when(sid < NS - 1)
    def _():
        @pl.loop(0, RPT // ZB)
        def _(k):
            pltpu.sync_copy(zbuf, acc.at[pl.ds(sid * RPT + k * ZB, ZB)])

    @pl.when(sid == NS - 1)
    def _():
        @pl.loop(0, RPT_LAST // ZB)
        def _(k):
            pltpu.sync_copy(zbuf, acc.at[pl.ds((NS - 1) * RPT + k * ZB, ZB)])

    plsc.subcore_barrier()

    def issue_gather(c, b):
        pltpu.async_copy(x_hbm.at[sidx.at[c]], bufx.at[b], sg[b])

    def wait_gather(b):
        pltpu.make_async_copy(x_hbm.at[sidx.at[0]], bufx.at[b], sg[b]).wait()

    for h in range(H1):
        pltpu.sync_copy(src_hbm.at[sid, h], sidx)
        pltpu.sync_copy(dst_hbm.at[sid, h], didx)
        issue_gather(0, 0)
        issue_gather(1, 1)

        @pl.loop(0, HC // 2)
        def _(p):
            for b in range(2):
                c = p * 2 + b
                wait_gather(b)
                pltpu.sync_copy(bufx.at[b], acc.at[didx.at[c]], add=True)

                @pl.when(p < HC // 2 - 1)
                def _():
                    issue_gather(c + 2, b)

    plsc.subcore_barrier()

    @pl.when(sid < NS - 1)
    def _():
        r = sid * RPT
        pltpu.sync_copy(acc.at[pl.ds(r, RPT)], out_hbm.at[pl.ds(r, RPT)])

    @pl.when(sid == NS - 1)
    def _():
        r = (NS - 1) * RPT
        pltpu.sync_copy(acc.at[pl.ds(r, RPT_LAST)],
                        out_hbm.at[pl.ds(r, RPT_LAST)])


def _linear_body(x_ref, s_ref, w_ref, b_ref, a_ref, bb_ref):
    w1 = w_ref[0:D, :]
    w2 = w_ref[D:2 * D, :]
    a_ref[...] = jnp.dot(x_ref[...], w1,
                         preferred_element_type=jnp.float32) + b_ref[...]
    bb_ref[...] = jnp.dot(s_ref[...], w2,
                          preferred_element_type=jnp.float32)


_ROWS_BLK = 1000


def _linear_tc(x, s, W, b):
    grid = (N_NODES // _ROWS_BLK,)
    blk = pl.BlockSpec((_ROWS_BLK, D), lambda i: (i, 0))
    return pl.pallas_call(
        _linear_body,
        grid=grid,
        in_specs=[blk, blk,
                  pl.BlockSpec((2 * D, D), lambda i: (0, 0)),
                  pl.BlockSpec((1, D), lambda i: (0, 0))],
        out_specs=[blk, blk],
        out_shape=[jax.ShapeDtypeStruct((N_NODES, D), jnp.float32)] * 2,
    )(x, s, W, b.reshape(1, D))


@functools.partial(
    pl.kernel,
    out_type=jax.ShapeDtypeStruct((N_EDGES, D), jnp.float32),
    mesh=_mesh,
    scratch_types=[
        pltpu.VMEM((EPW,), jnp.int32),
        pltpu.VMEM((EPW,), jnp.int32),
        pltpu.VMEM((2, CH, D), jnp.float32),
        pltpu.VMEM((2, CH, D), jnp.float32),
        pltpu.VMEM((2, CH, D), jnp.float32),
        pltpu.SemaphoreType.DMA,
        pltpu.SemaphoreType.DMA,
        pltpu.SemaphoreType.DMA,
        pltpu.SemaphoreType.DMA,
    ],
)
def _edge_combine_sc(a_hbm, b_hbm, src_hbm, dst_hbm, out_hbm,
                     sidx, didx, bufa, bufb, bufo, sg0, sg1, so0, so1):
    cid = lax.axis_index("c")
    sid = lax.axis_index("s")
    wid = sid * NC + cid
    ebase = wid * EPW
    sg = (sg0, sg1)
    so = (so0, so1)

    pltpu.sync_copy(src_hbm.at[pl.ds(ebase, EPW)], sidx)
    pltpu.sync_copy(dst_hbm.at[pl.ds(ebase, EPW)], didx)

    def issue_gathers(c, b):
        pltpu.async_copy(a_hbm.at[sidx.at[pl.ds(c * CH, CH)]], bufa.at[b],
                         sg[b])
        pltpu.async_copy(b_hbm.at[didx.at[pl.ds(c * CH, CH)]], bufb.at[b],
                         sg[b])

    def wait_gathers(b):
        pltpu.make_async_copy(a_hbm.at[sidx.at[pl.ds(0, CH)]], bufa.at[b],
                              sg[b]).wait()
        pltpu.make_async_copy(b_hbm.at[didx.at[pl.ds(0, CH)]], bufb.at[b],
                              sg[b]).wait()

    def do_add(b):
        @pl.loop(0, CH)
        def _(r):
            for j in range(D // 16):
                sl = (b, pl.ds(r, 1), pl.ds(j * 16, 16))
                bufo.at[sl][...] = bufa.at[sl][...] + bufb.at[sl][...]

    def issue_out(c, b):
        pltpu.async_copy(bufo.at[b], out_hbm.at[pl.ds(ebase + c * CH, CH)],
                         so[b])

    def wait_out(b):
        pltpu.make_async_copy(bufo.at[b], out_hbm.at[pl.ds(ebase, CH)],
                              so[b]).wait()

    issue_gathers(0, 0)
    issue_gathers(1, 1)

    @pl.loop(0, (CHUNKS - 1) // 2)
    def _(p):
        for b in range(2):
            c = p * 2 + b
            wait_gathers(b)

            @pl.when(p > 0)
            def _():
                wait_out(b)

            do_add(b)
            if b == 0:
                issue_gathers(c + 2, b)
            else:
                @pl.when(p < (CHUNKS - 1) // 2 - 1)
                def _():
                    issue_gathers(c + 2, b)
            issue_out(c, b)

    wait_gathers(0)
    wait_out(0)
    do_add(0)
    issue_out(CHUNKS - 1, 0)
    wait_out(1)
    wait_out(0)


def kernel(x, edge_index, W, b):
    src = edge_index[0].astype(jnp.int32)
    dst = edge_index[1].astype(jnp.int32)
    s = _segment_sum_sc(x, src.reshape(NS, H1, HC, CH),
                        dst.reshape(NS, H1, HC, CH))
    a, bb = _linear_tc(x, s, W, b)
    return _edge_combine_sc(a, bb, src, dst)
